# Initial kernel scaffold; baseline (speedup 1.0000x reference)
import functools

import jax
import jax.numpy as jnp
from jax import lax
from jax.experimental import pallas as pl
from jax.experimental.pallas import tpu as pltpu
from jax.experimental.pallas import tpu_sc as plsc

_VOCAB = 100000
_D = 64
_BATCH = 4096
_SEQ = 200

_NW = 32
_ROWS = _BATCH * _SEQ
_RPW = _ROWS // _NW
_CHUNK = 128
_NCHUNK = _RPW // _CHUNK


def _build(interpret=False):
  mesh = plsc.VectorSubcoreMesh(core_axis_name="c", subcore_axis_name="s")
  nc = 2

  @functools.partial(
      pl.kernel,
      out_type=jax.ShapeDtypeStruct((_ROWS, _D), jnp.float32),
      mesh=mesh,
      scratch_types=[
          pltpu.VMEM((_NCHUNK, _CHUNK), jnp.int32),
          pltpu.VMEM((_SEQ + _CHUNK, _D), jnp.float32),
          pltpu.VMEM((_CHUNK, _D), jnp.float32),
          pltpu.SemaphoreType.DMA,
      ],
      interpret=interpret,
  )
  def k(table_hbm, idx_hbm, pos_hbm, out_hbm, idx_v, pos_v, buf, sem):
    wid = lax.axis_index("s") * nc + lax.axis_index("c")
    base = wid * _RPW

    pltpu.sync_copy(idx_hbm.at[wid], idx_v)
    pltpu.sync_copy(pos_hbm, pos_v)

    def body(g, carry):
      off = lax.rem(g * _CHUNK, _SEQ)
      pltpu.sync_copy(pos_v.at[pl.ds(off, _CHUNK)], buf)
      pltpu.async_copy(table_hbm.at[idx_v.at[g]], buf, sem, add=True).wait()
      pltpu.sync_copy(buf, out_hbm.at[pl.ds(base + g * _CHUNK, _CHUNK)])
      return carry

    lax.fori_loop(0, _NCHUNK, body, 0)

  return k


_kernel_call = _build()


def kernel(x, word_emb, pos_emb):
  xf = x.astype(jnp.int32).reshape(_NW, _NCHUNK, _CHUNK)
  pos = pos_emb[:_SEQ]
  pos_ext = jnp.concatenate([pos, pos[:_CHUNK]], axis=0)
  out = _kernel_call(word_emb, xf, pos_ext)
  return out.reshape(_BATCH, _SEQ, _D)


# speedup vs baseline: 3.3216x; 3.3216x over previous
import functools

import jax
import jax.numpy as jnp
from jax import lax
from jax.experimental import pallas as pl
from jax.experimental.pallas import tpu as pltpu
from jax.experimental.pallas import tpu_sc as plsc

_VOCAB = 100000
_D = 64
_BATCH = 4096
_SEQ = 200

_NW = 32
_ROWS = _BATCH * _SEQ
_RPW = _ROWS // _NW
_CHUNK = 128
_NCHUNK = _RPW // _CHUNK


def _build(interpret=False):
  mesh = plsc.VectorSubcoreMesh(core_axis_name="c", subcore_axis_name="s")
  nc = 2

  @functools.partial(
      pl.kernel,
      out_type=jax.ShapeDtypeStruct((_ROWS, _D), jnp.float32),
      mesh=mesh,
      scratch_types=[
          pltpu.VMEM((_NCHUNK, _CHUNK), jnp.int32),
          pltpu.VMEM_SHARED((_SEQ + _CHUNK, _D), jnp.float32),
          pltpu.VMEM((_CHUNK, _D), jnp.float32),
          pltpu.SemaphoreType.DMA,
      ],
      compiler_params=pltpu.CompilerParams(use_tc_tiling_on_sc=False),
      interpret=interpret,
  )
  def k(table_hbm, idx_hbm, pos_hbm, out_hbm, idx_v, pos_v, buf, sem):
    sid = lax.axis_index("s")
    wid = sid * nc + lax.axis_index("c")
    base = wid * _RPW

    pltpu.sync_copy(idx_hbm.at[wid], idx_v)
    @pl.when(sid == 0)
    def _():
      pltpu.sync_copy(pos_hbm, pos_v)
    plsc.subcore_barrier()

    def body(g, carry):
      off = lax.rem(g * _CHUNK, _SEQ)
      pltpu.sync_copy(pos_v.at[pl.ds(off, _CHUNK)], buf)
      pltpu.async_copy(table_hbm.at[idx_v.at[g]], buf, sem, add=True).wait()
      pltpu.sync_copy(buf, out_hbm.at[pl.ds(base + g * _CHUNK, _CHUNK)])
      return carry

    lax.fori_loop(0, _NCHUNK, body, 0)

  return k


_kernel_call = _build()


def kernel(x, word_emb, pos_emb):
  xf = x.astype(jnp.int32).reshape(_NW, _NCHUNK, _CHUNK)
  pos = pos_emb[:_SEQ]
  pos_ext = jnp.concatenate([pos, pos[:_CHUNK]], axis=0)
  out = _kernel_call(word_emb, xf, pos_ext)
  return out.reshape(_BATCH, _SEQ, _D)


# speedup vs baseline: 4.1299x; 1.2433x over previous
import functools

import jax
import jax.numpy as jnp
from jax import lax
from jax.experimental import pallas as pl
from jax.experimental.pallas import tpu as pltpu
from jax.experimental.pallas import tpu_sc as plsc

_VOCAB = 100000
_D = 64
_BATCH = 4096
_SEQ = 200

_NW = 32
_ROWS = _BATCH * _SEQ
_RPW = _ROWS // _NW
_CHUNK = 128
_NCHUNK = _RPW // _CHUNK
_NBUF = 4


def _build(interpret=False):
  mesh = plsc.VectorSubcoreMesh(core_axis_name="c", subcore_axis_name="s")
  nc = 2

  @functools.partial(
      pl.kernel,
      out_type=jax.ShapeDtypeStruct((_ROWS, _D), jnp.float32),
      mesh=mesh,
      scratch_types=[
          pltpu.VMEM((_NCHUNK, _CHUNK), jnp.int32),
          pltpu.VMEM_SHARED((_SEQ + _CHUNK, _D), jnp.float32),
          pltpu.VMEM((_NBUF, _CHUNK, _D), jnp.float32),
      ] + [pltpu.SemaphoreType.DMA] * (2 * _NBUF),
      compiler_params=pltpu.CompilerParams(use_tc_tiling_on_sc=False),
      interpret=interpret,
  )
  def k(table_hbm, idx_hbm, pos_hbm, out_hbm, idx_v, pos_v, bufs, *sems):
    gsems = sems[:_NBUF]
    wsems = sems[_NBUF:]
    sid = lax.axis_index("s")
    wid = sid * nc + lax.axis_index("c")
    base = wid * _RPW

    pltpu.sync_copy(idx_hbm.at[wid], idx_v)
    @pl.when(sid == 0)
    def _():
      pltpu.sync_copy(pos_hbm, pos_v)
    plsc.subcore_barrier()

    def stage(g, b):
      off = lax.rem(g * _CHUNK, _SEQ)
      pltpu.sync_copy(pos_v.at[pl.ds(off, _CHUNK)], bufs.at[b])
      pltpu.async_copy(table_hbm.at[idx_v.at[g]], bufs.at[b], gsems[b], add=True)

    for b in range(_NBUF):
      stage(b, b)

    def body(i, carry):
      g0 = i * _NBUF
      for b in range(_NBUF):
        g = g0 + b
        pltpu.make_async_copy(
            out_hbm.at[pl.ds(base, _CHUNK)], bufs.at[b], gsems[b]).wait()
        pltpu.async_copy(
            bufs.at[b], out_hbm.at[pl.ds(base + g * _CHUNK, _CHUNK)], wsems[b])

        @pl.when(g + _NBUF < _NCHUNK)
        def _():
          pltpu.make_async_copy(
              bufs.at[b], out_hbm.at[pl.ds(base, _CHUNK)], wsems[b]).wait()
          stage(g + _NBUF, b)

      return carry

    lax.fori_loop(0, _NCHUNK // _NBUF, body, 0)

    for b in range(_NBUF):
      pltpu.make_async_copy(
          bufs.at[b], out_hbm.at[pl.ds(base, _CHUNK)], wsems[b]).wait()

  return k


_kernel_call = _build()


def kernel(x, word_emb, pos_emb):
  xf = x.astype(jnp.int32).reshape(_NW, _NCHUNK, _CHUNK)
  pos = pos_emb[:_SEQ]
  pos_ext = jnp.concatenate([pos, pos[:_CHUNK]], axis=0)
  out = _kernel_call(word_emb, xf, pos_ext)
  return out.reshape(_BATCH, _SEQ, _D)


# speedup vs baseline: 4.1328x; 1.0007x over previous
import functools

import jax
import jax.numpy as jnp
from jax import lax
from jax.experimental import pallas as pl
from jax.experimental.pallas import tpu as pltpu
from jax.experimental.pallas import tpu_sc as plsc

_VOCAB = 100000
_D = 64
_BATCH = 4096
_SEQ = 200

_NW = 32
_ROWS = _BATCH * _SEQ
_RPW = _ROWS // _NW
_CHUNK = 128
_NCHUNK = _RPW // _CHUNK
_NBUF = 8


def _build(interpret=False):
  mesh = plsc.VectorSubcoreMesh(core_axis_name="c", subcore_axis_name="s")
  nc = 2

  @functools.partial(
      pl.kernel,
      out_type=jax.ShapeDtypeStruct((_ROWS, _D), jnp.float32),
      mesh=mesh,
      scratch_types=[
          pltpu.VMEM((_NCHUNK, _CHUNK), jnp.int32),
          pltpu.VMEM_SHARED((_SEQ + _CHUNK, _D), jnp.float32),
          pltpu.VMEM((_NBUF, _CHUNK, _D), jnp.float32),
      ] + [pltpu.SemaphoreType.DMA] * (2 * _NBUF),
      compiler_params=pltpu.CompilerParams(use_tc_tiling_on_sc=False),
      interpret=interpret,
  )
  def k(table_hbm, idx_hbm, pos_hbm, out_hbm, idx_v, pos_v, bufs, *sems):
    gsems = sems[:_NBUF]
    wsems = sems[_NBUF:]
    sid = lax.axis_index("s")
    wid = sid * nc + lax.axis_index("c")
    base = wid * _RPW

    pltpu.sync_copy(idx_hbm.at[wid], idx_v)
    @pl.when(sid == 0)
    def _():
      pltpu.sync_copy(pos_hbm, pos_v)
    plsc.subcore_barrier()

    def stage(g, b):
      off = lax.rem(g * _CHUNK, _SEQ)
      pltpu.sync_copy(pos_v.at[pl.ds(off, _CHUNK)], bufs.at[b])
      pltpu.async_copy(table_hbm.at[idx_v.at[g]], bufs.at[b], gsems[b], add=True)

    for b in range(_NBUF):
      stage(b, b)

    def body(i, carry):
      g0 = i * _NBUF
      for b in range(_NBUF):
        g = g0 + b
        pltpu.make_async_copy(
            out_hbm.at[pl.ds(base, _CHUNK)], bufs.at[b], gsems[b]).wait()
        pltpu.async_copy(
            bufs.at[b], out_hbm.at[pl.ds(base + g * _CHUNK, _CHUNK)], wsems[b])

        @pl.when(g + _NBUF < _NCHUNK)
        def _():
          pltpu.make_async_copy(
              bufs.at[b], out_hbm.at[pl.ds(base, _CHUNK)], wsems[b]).wait()
          stage(g + _NBUF, b)

      return carry

    lax.fori_loop(0, _NCHUNK // _NBUF, body, 0)

    for b in range(_NBUF):
      pltpu.make_async_copy(
          bufs.at[b], out_hbm.at[pl.ds(base, _CHUNK)], wsems[b]).wait()

  return k


_kernel_call = _build()


def kernel(x, word_emb, pos_emb):
  xf = x.astype(jnp.int32).reshape(_NW, _NCHUNK, _CHUNK)
  pos = pos_emb[:_SEQ]
  pos_ext = jnp.concatenate([pos, pos[:_CHUNK]], axis=0)
  out = _kernel_call(word_emb, xf, pos_ext)
  return out.reshape(_BATCH, _SEQ, _D)
